# baseline (speedup 1.0000x reference)
import functools

import jax
import jax.numpy as jnp
from jax import lax
from jax.experimental import pallas as pl
from jax.experimental.pallas import tpu as pltpu
from jax.experimental.pallas import tpu_sc as plsc

N_NODES = 10000
N_EDGES = 320000
NODE_LAT = 128
EDGE_LAT = 128

NC = 2
NS = 16
NW = NC * NS
EPW = N_EDGES // NW
GN = 80
NCHUNK = EPW // GN
NPAD = 10240
ROWS_PER_TILE = NPAD // NS
ZCH = 64
NZ = ROWS_PER_TILE // ZCH
CW = 128

BE = 1280
BN = 2000

@functools.cache
def _sc_mesh():
    return plsc.VectorSubcoreMesh(
        core_axis_name="c", subcore_axis_name="s",
        num_cores=NC, num_subcores=NS)


def _swish(x):
    return x * lax.logistic(x)


def _swish_bf(x):
    xb = x.astype(jnp.bfloat16)
    return xb * lax.logistic(xb)


def _ln_apply(x, scale, shift):
    ones = jnp.ones((x.shape[1], 1), jnp.float32)
    s1 = jnp.dot(x, ones, preferred_element_type=jnp.float32)
    s2 = jnp.dot(x * x, ones, preferred_element_type=jnp.float32)
    inv_d = 1.0 / x.shape[1]
    mu = s1 * inv_d
    var = s2 * inv_d - mu * mu
    inv = lax.rsqrt(var + 1e-5)
    return (x - mu) * inv * (1.0 + scale) + shift


def _tau_body(*refs):
    tau = refs[0][...]
    n_ln = (len(refs) - 1) // 8
    for i in range(n_ln):
        wc, bc, ws, bs, wb, bb = refs[1 + 6 * i:1 + 6 * i + 6]
        h = _swish(tau * wc[...] + bc[...])
        refs[1 + 6 * n_ln + 2 * i][...] = (
            jnp.dot(h, ws[...], preferred_element_type=jnp.float32) + bs[...])
        refs[1 + 6 * n_ln + 2 * i + 1][...] = (
            jnp.dot(h, wb[...], preferred_element_type=jnp.float32) + bb[...])


def _embed_body(ef_ref, w1_ref, b1_ref, w2_ref, b2_ref, sc_ref, sh_ref,
                out_ref):
    ef = ef_ref[...]
    w1 = w1_ref[...]
    h = b1_ref[...]
    for i in range(ef.shape[1]):
        h = h + ef[:, i:i + 1] * w1[i:i + 1, :]
    y = jnp.dot(_swish_bf(h), w2_ref[...],
                preferred_element_type=jnp.float32) + b2_ref[...]
    out_ref[...] = _ln_apply(y, sc_ref[...], sh_ref[...])


def _edge_body(el_ref, s_ref, r_ref, w1e_ref, w1s_ref, w1r_ref,
               b1_ref, w2_ref, b2_ref, sc_ref, sh_ref, out_ref):
    el = el_ref[...]
    h = (jnp.dot(el.astype(jnp.bfloat16), w1e_ref[...],
                 preferred_element_type=jnp.float32)
         + jnp.dot(s_ref[...].astype(jnp.bfloat16), w1s_ref[...],
                   preferred_element_type=jnp.float32)
         + jnp.dot(r_ref[...].astype(jnp.bfloat16), w1r_ref[...],
                   preferred_element_type=jnp.float32)
         + b1_ref[...])
    y = jnp.dot(_swish_bf(h), w2_ref[...],
                preferred_element_type=jnp.float32) + b2_ref[...]
    out_ref[...] = el + _ln_apply(y, sc_ref[...], sh_ref[...])


def _node_body(nl_ref, sum0_ref, sum1_ref, cnt0_ref, cnt1_ref,
               w1n_ref, w1a_ref, b1_ref, w2_ref, b2_ref, sc_ref, sh_ref,
               out_ref):
    nl = nl_ref[...]
    counts = cnt0_ref[:, 0:1] + cnt1_ref[:, 0:1]
    counts = jnp.maximum(counts, 1.0)
    agg = (sum0_ref[...] + sum1_ref[...]) / counts
    h = (jnp.dot(nl.astype(jnp.bfloat16), w1n_ref[...],
                 preferred_element_type=jnp.float32)
         + jnp.dot(agg.astype(jnp.bfloat16), w1a_ref[...],
                   preferred_element_type=jnp.float32)
         + b1_ref[...])
    y = jnp.dot(_swish_bf(h), w2_ref[...],
                preferred_element_type=jnp.float32) + b2_ref[...]
    out_ref[...] = nl + _ln_apply(y, sc_ref[...], sh_ref[...])


def _full(shape):
    return pl.BlockSpec(shape, lambda i: (0,) * len(shape))


def _rows(block_rows, cols):
    return pl.BlockSpec((block_rows, cols), lambda i: (i, 0))


def _tc_tau(tau11, lns):
    args = [tau11]
    for p in lns:
        args += [p["Wc"].reshape(1, -1), p["bc"].reshape(1, -1), p["Ws"],
                 p["bs"].reshape(1, -1), p["Wb"], p["bb"].reshape(1, -1)]
    outs = pl.pallas_call(
        _tau_body,
        out_shape=[jax.ShapeDtypeStruct((1, 128), jnp.float32)
                   for _ in range(2 * len(lns))],
    )(*args)
    return [(outs[2 * i], outs[2 * i + 1]) for i in range(len(lns))]


def _tc_embed(ef, mlp, ss):
    (w1, b1), (w2, b2) = mlp
    grid = (N_EDGES // BE,)
    return pl.pallas_call(
        _embed_body,
        grid=grid,
        in_specs=[_rows(BE, 4), _full((4, 256)), _full((1, 256)),
                  _full((256, 128)), _full((1, 128)),
                  _full((1, 128)), _full((1, 128))],
        out_specs=_rows(BE, 128),
        out_shape=jax.ShapeDtypeStruct((N_EDGES, 128), jnp.float32),
    )(ef, w1, b1.reshape(1, -1), w2.astype(jnp.bfloat16),
      b2.reshape(1, -1), ss[0], ss[1])


def _tc_edge(el, s, r, mlp, ss):
    (w1, b1), (w2, b2) = mlp
    grid = (N_EDGES // BE,)
    return pl.pallas_call(
        _edge_body,
        grid=grid,
        in_specs=[_rows(BE, 128), _rows(BE, 128), _rows(BE, 128),
                  _full((128, 256)), _full((128, 256)), _full((128, 256)),
                  _full((1, 256)), _full((256, 128)), _full((1, 128)),
                  _full((1, 128)), _full((1, 128))],
        out_specs=_rows(BE, 128),
        out_shape=jax.ShapeDtypeStruct((N_EDGES, 128), jnp.float32),
    )(el, s, r, w1[:128].astype(jnp.bfloat16),
      w1[128:256].astype(jnp.bfloat16), w1[256:].astype(jnp.bfloat16),
      b1.reshape(1, -1), w2.astype(jnp.bfloat16), b2.reshape(1, -1),
      ss[0], ss[1])


def _tc_node(nl, sums, cnts, mlp, ss):
    (w1, b1), (w2, b2) = mlp
    grid = (N_NODES // BN,)
    return pl.pallas_call(
        _node_body,
        grid=grid,
        in_specs=[_rows(BN, 128), _rows(BN, 128), _rows(BN, 128),
                  _rows(BN, CW), _rows(BN, CW),
                  _full((128, 256)), _full((128, 256)), _full((1, 256)),
                  _full((256, 128)), _full((1, 128)),
                  _full((1, 128)), _full((1, 128))],
        out_specs=_rows(BN, 128),
        out_shape=jax.ShapeDtypeStruct((N_NODES, 128), jnp.float32),
    )(nl, sums[0], sums[1], cnts[0], cnts[1],
      w1[:128].astype(jnp.bfloat16), w1[128:].astype(jnp.bfloat16),
      b1.reshape(1, -1), w2.astype(jnp.bfloat16), b2.reshape(1, -1),
      ss[0], ss[1])


def _sc_gather_body(tbl, send3d, recv3d, s_out, r_out,
                    sidx, ridx, bs0, bs1, br0, br1,
                    sem_s0, sem_s1, sem_r0, sem_r1):
    wid = lax.axis_index("s") * NC + lax.axis_index("c")
    pltpu.sync_copy(send3d.at[wid], sidx)
    pltpu.sync_copy(recv3d.at[wid], ridx)
    base = wid * EPW

    def issue(j, bs, br, ss, sr):
        pltpu.async_copy(tbl.at[sidx.at[j]], bs, ss)
        pltpu.async_copy(tbl.at[ridx.at[j]], br, sr)

    def drain(j, bs, br, ss, sr):
        pltpu.make_async_copy(tbl.at[sidx.at[j]], bs, ss).wait()
        pltpu.sync_copy(bs, s_out.at[pl.ds(base + j * GN, GN)])
        pltpu.make_async_copy(tbl.at[ridx.at[j]], br, sr).wait()
        pltpu.sync_copy(br, r_out.at[pl.ds(base + j * GN, GN)])

    issue(0, bs0, br0, sem_s0, sem_r0)

    def body(i, carry):
        j0 = 2 * i
        issue(j0 + 1, bs1, br1, sem_s1, sem_r1)
        drain(j0, bs0, br0, sem_s0, sem_r0)
        issue(j0 + 2, bs0, br0, sem_s0, sem_r0)
        drain(j0 + 1, bs1, br1, sem_s1, sem_r1)
        return carry

    lax.fori_loop(0, (NCHUNK - 1) // 2, body, 0)
    drain(NCHUNK - 1, bs0, br0, sem_s0, sem_r0)


@functools.cache
def _sc_gather():
    return pl.kernel(
        _sc_gather_body,
        mesh=_sc_mesh(),
        out_type=[jax.ShapeDtypeStruct((N_EDGES, 128), jnp.float32),
                  jax.ShapeDtypeStruct((N_EDGES, 128), jnp.float32)],
        scratch_types=[pltpu.VMEM((NCHUNK, GN), jnp.int32),
                       pltpu.VMEM((NCHUNK, GN), jnp.int32),
                       pltpu.VMEM((GN, 128), jnp.float32),
                       pltpu.VMEM((GN, 128), jnp.float32),
                       pltpu.VMEM((GN, 128), jnp.float32),
                       pltpu.VMEM((GN, 128), jnp.float32),
                       pltpu.SemaphoreType.DMA,
                       pltpu.SemaphoreType.DMA,
                       pltpu.SemaphoreType.DMA,
                       pltpu.SemaphoreType.DMA],
    )


def _sc_scatter_body(edge_hbm, recv3d, zeros128,
                     sums_out, idx_v, rows_v, st_v, acc_sh):
    cid = lax.axis_index("c")
    sid = lax.axis_index("s")
    wid = sid * NC + cid
    r0 = sid * ROWS_PER_TILE
    pltpu.sync_copy(zeros128, st_v)

    def zbody(k, carry):
        pltpu.sync_copy(st_v, acc_sh.at[pl.ds(r0 + k * ZCH, ZCH)])
        return carry

    lax.fori_loop(0, NZ, zbody, 0)
    pltpu.sync_copy(recv3d.at[wid], idx_v)
    plsc.subcore_barrier()

    def body(j, carry):
        pltpu.sync_copy(edge_hbm.at[pl.ds(wid * EPW + j * GN, GN)], rows_v)
        pltpu.sync_copy(rows_v, acc_sh.at[idx_v.at[j]], add=True)
        return carry

    lax.fori_loop(0, NCHUNK, body, 0)
    plsc.subcore_barrier()

    def obody(k, carry):
        pltpu.sync_copy(acc_sh.at[pl.ds(r0 + k * ZCH, ZCH)], st_v)
        pltpu.sync_copy(st_v, sums_out.at[cid, pl.ds(r0 + k * ZCH, ZCH)])
        return carry

    lax.fori_loop(0, NZ, obody, 0)


@functools.cache
def _sc_scatter():
    return pl.kernel(
        _sc_scatter_body,
        mesh=_sc_mesh(),
        out_type=jax.ShapeDtypeStruct((NC, NPAD, 128), jnp.float32),
        scratch_types=[pltpu.VMEM((NCHUNK, GN), jnp.int32),
                       pltpu.VMEM((GN, 128), jnp.float32),
                       pltpu.VMEM((ZCH, 128), jnp.float32),
                       pltpu.VMEM_SHARED((NPAD, 128), jnp.float32)],
    )


def _sc_counts_body(recv3d, zerosc, onesc,
                    cnts_out, idx_v, ones_v, stc_v, cnt_sh):
    cid = lax.axis_index("c")
    sid = lax.axis_index("s")
    wid = sid * NC + cid
    r0 = sid * ROWS_PER_TILE
    pltpu.sync_copy(zerosc, stc_v)

    def zbody(k, carry):
        pltpu.sync_copy(stc_v, cnt_sh.at[pl.ds(r0 + k * ZCH, ZCH)])
        return carry

    lax.fori_loop(0, NZ, zbody, 0)
    pltpu.sync_copy(onesc, ones_v)
    pltpu.sync_copy(recv3d.at[wid], idx_v)
    plsc.subcore_barrier()

    def body(j, carry):
        pltpu.sync_copy(ones_v, cnt_sh.at[idx_v.at[j]], add=True)
        return carry

    lax.fori_loop(0, NCHUNK, body, 0)
    plsc.subcore_barrier()

    def obody(k, carry):
        pltpu.sync_copy(cnt_sh.at[pl.ds(r0 + k * ZCH, ZCH)], stc_v)
        pltpu.sync_copy(stc_v, cnts_out.at[cid, pl.ds(r0 + k * ZCH, ZCH)])
        return carry

    lax.fori_loop(0, NZ, obody, 0)


@functools.cache
def _sc_counts():
    return pl.kernel(
        _sc_counts_body,
        mesh=_sc_mesh(),
        out_type=jax.ShapeDtypeStruct((NC, NPAD, CW), jnp.float32),
        scratch_types=[pltpu.VMEM((NCHUNK, GN), jnp.int32),
                       pltpu.VMEM((GN, CW), jnp.float32),
                       pltpu.VMEM((ZCH, CW), jnp.float32),
                       pltpu.VMEM_SHARED((NPAD, CW), jnp.float32)],
    )


def kernel(rnode_features, edge_features, senders, receivers, tau, params):
    tau11 = tau.astype(jnp.float32).reshape(1, 1)
    node_lat = rnode_features[:, 0, :]
    send3d = senders.reshape(NW, NCHUNK, GN)
    recv3d = receivers.reshape(NW, NCHUNK, GN)

    zeros128 = jnp.zeros((ZCH, 128), jnp.float32)
    zerosc = jnp.zeros((ZCH, CW), jnp.float32)
    onesc = jnp.ones((GN, CW), jnp.float32)

    lns = [params["embed_ln"]]
    for sp in params["steps"]:
        lns += [sp["edge_ln"], sp["node_ln"]]
    ss = _tc_tau(tau11, lns)

    edge_lat = _tc_embed(edge_features.astype(jnp.float32),
                         params["embed_mlp"], ss[0])
    cnts = _sc_counts()(recv3d, zerosc, onesc)

    for i, sp in enumerate(params["steps"]):
        s_buf, r_buf = _sc_gather()(node_lat, send3d, recv3d)
        edge_lat = _tc_edge(edge_lat, s_buf, r_buf,
                            sp["edge_mlp"], ss[1 + 2 * i])
        sums = _sc_scatter()(edge_lat, recv3d, zeros128)
        node_lat = _tc_node(node_lat, sums, cnts,
                            sp["node_mlp"], ss[2 + 2 * i])

    return node_lat[:, None, :]


# speedup vs baseline: 1.0872x; 1.0872x over previous
import functools

import jax
import jax.numpy as jnp
from jax import lax
from jax.experimental import pallas as pl
from jax.experimental.pallas import tpu as pltpu
from jax.experimental.pallas import tpu_sc as plsc

N_NODES = 10000
N_EDGES = 320000
NODE_LAT = 128
EDGE_LAT = 128

NC = 2
NS = 16
NW = NC * NS
EPW = N_EDGES // NW
GN = 80
NCHUNK = EPW // GN
NPAD = 10240
ROWS_PER_TILE = NPAD // NS
ZCH = 64
NZ = ROWS_PER_TILE // ZCH
CW = 128

BE = 2560
BN = 2000

@functools.cache
def _sc_mesh():
    return plsc.VectorSubcoreMesh(
        core_axis_name="c", subcore_axis_name="s",
        num_cores=NC, num_subcores=NS)


def _swish(x):
    return x * lax.logistic(x)


def _swish_bf(x):
    xb = x.astype(jnp.bfloat16)
    return xb * lax.logistic(xb)


def _ln_apply(x, scale, shift):
    ones = jnp.ones((x.shape[1], 1), jnp.float32)
    s1 = jnp.dot(x, ones, preferred_element_type=jnp.float32)
    s2 = jnp.dot(x * x, ones, preferred_element_type=jnp.float32)
    inv_d = 1.0 / x.shape[1]
    mu = s1 * inv_d
    var = s2 * inv_d - mu * mu
    inv = lax.rsqrt(var + 1e-5)
    return (x - mu) * inv * (1.0 + scale) + shift


def _tau_body(*refs):
    tau = refs[0][...]
    n_ln = (len(refs) - 1) // 8
    for i in range(n_ln):
        wc, bc, ws, bs, wb, bb = refs[1 + 6 * i:1 + 6 * i + 6]
        h = _swish(tau * wc[...] + bc[...])
        refs[1 + 6 * n_ln + 2 * i][...] = (
            jnp.dot(h, ws[...], preferred_element_type=jnp.float32) + bs[...])
        refs[1 + 6 * n_ln + 2 * i + 1][...] = (
            jnp.dot(h, wb[...], preferred_element_type=jnp.float32) + bb[...])


def _embed_body(ef_ref, w1_ref, b1_ref, w2_ref, b2_ref, sc_ref, sh_ref,
                out_ref):
    ef = ef_ref[...]
    w1 = w1_ref[...]
    h = b1_ref[...]
    for i in range(ef.shape[1]):
        h = h + ef[:, i:i + 1] * w1[i:i + 1, :]
    y = jnp.dot(_swish_bf(h), w2_ref[...],
                preferred_element_type=jnp.float32) + b2_ref[...]
    out_ref[...] = _ln_apply(y, sc_ref[...], sh_ref[...])


def _edge_body(el_ref, s_ref, r_ref, w1e_ref, w1s_ref, w1r_ref,
               b1_ref, w2_ref, b2_ref, sc_ref, sh_ref, out_ref):
    el = el_ref[...]
    h = (jnp.dot(el.astype(jnp.bfloat16), w1e_ref[...],
                 preferred_element_type=jnp.float32)
         + jnp.dot(s_ref[...].astype(jnp.bfloat16), w1s_ref[...],
                   preferred_element_type=jnp.float32)
         + jnp.dot(r_ref[...].astype(jnp.bfloat16), w1r_ref[...],
                   preferred_element_type=jnp.float32)
         + b1_ref[...])
    y = jnp.dot(_swish_bf(h), w2_ref[...],
                preferred_element_type=jnp.float32) + b2_ref[...]
    out_ref[...] = el + _ln_apply(y, sc_ref[...], sh_ref[...])


def _node_body(nl_ref, sum0_ref, sum1_ref, cnt0_ref, cnt1_ref,
               w1n_ref, w1a_ref, b1_ref, w2_ref, b2_ref, sc_ref, sh_ref,
               out_ref):
    nl = nl_ref[...]
    counts = cnt0_ref[:, 0:1] + cnt1_ref[:, 0:1]
    counts = jnp.maximum(counts, 1.0)
    agg = (sum0_ref[...] + sum1_ref[...]) / counts
    h = (jnp.dot(nl.astype(jnp.bfloat16), w1n_ref[...],
                 preferred_element_type=jnp.float32)
         + jnp.dot(agg.astype(jnp.bfloat16), w1a_ref[...],
                   preferred_element_type=jnp.float32)
         + b1_ref[...])
    y = jnp.dot(_swish_bf(h), w2_ref[...],
                preferred_element_type=jnp.float32) + b2_ref[...]
    out_ref[...] = nl + _ln_apply(y, sc_ref[...], sh_ref[...])


def _full(shape):
    return pl.BlockSpec(shape, lambda i: (0,) * len(shape))


def _rows(block_rows, cols):
    return pl.BlockSpec((block_rows, cols), lambda i: (i, 0))


def _tc_tau(tau11, lns):
    args = [tau11]
    for p in lns:
        args += [p["Wc"].reshape(1, -1), p["bc"].reshape(1, -1), p["Ws"],
                 p["bs"].reshape(1, -1), p["Wb"], p["bb"].reshape(1, -1)]
    outs = pl.pallas_call(
        _tau_body,
        out_shape=[jax.ShapeDtypeStruct((1, 128), jnp.float32)
                   for _ in range(2 * len(lns))],
    )(*args)
    return [(outs[2 * i], outs[2 * i + 1]) for i in range(len(lns))]


def _tc_embed(ef, mlp, ss):
    (w1, b1), (w2, b2) = mlp
    grid = (N_EDGES // BE,)
    return pl.pallas_call(
        _embed_body,
        grid=grid,
        in_specs=[_rows(BE, 4), _full((4, 256)), _full((1, 256)),
                  _full((256, 128)), _full((1, 128)),
                  _full((1, 128)), _full((1, 128))],
        out_specs=_rows(BE, 128),
        out_shape=jax.ShapeDtypeStruct((N_EDGES, 128), jnp.float32),
    )(ef, w1, b1.reshape(1, -1), w2.astype(jnp.bfloat16),
      b2.reshape(1, -1), ss[0], ss[1])


def _tc_edge(el, s, r, mlp, ss):
    (w1, b1), (w2, b2) = mlp
    grid = (N_EDGES // BE,)
    return pl.pallas_call(
        _edge_body,
        grid=grid,
        in_specs=[_rows(BE, 128), _rows(BE, 128), _rows(BE, 128),
                  _full((128, 256)), _full((128, 256)), _full((128, 256)),
                  _full((1, 256)), _full((256, 128)), _full((1, 128)),
                  _full((1, 128)), _full((1, 128))],
        out_specs=_rows(BE, 128),
        out_shape=jax.ShapeDtypeStruct((N_EDGES, 128), jnp.float32),
    )(el, s, r, w1[:128].astype(jnp.bfloat16),
      w1[128:256].astype(jnp.bfloat16), w1[256:].astype(jnp.bfloat16),
      b1.reshape(1, -1), w2.astype(jnp.bfloat16), b2.reshape(1, -1),
      ss[0], ss[1])


def _tc_node(nl, sums, cnts, mlp, ss):
    (w1, b1), (w2, b2) = mlp
    grid = (N_NODES // BN,)
    return pl.pallas_call(
        _node_body,
        grid=grid,
        in_specs=[_rows(BN, 128), _rows(BN, 128), _rows(BN, 128),
                  _rows(BN, CW), _rows(BN, CW),
                  _full((128, 256)), _full((128, 256)), _full((1, 256)),
                  _full((256, 128)), _full((1, 128)),
                  _full((1, 128)), _full((1, 128))],
        out_specs=_rows(BN, 128),
        out_shape=jax.ShapeDtypeStruct((N_NODES, 128), jnp.float32),
    )(nl, sums[0], sums[1], cnts[0], cnts[1],
      w1[:128].astype(jnp.bfloat16), w1[128:].astype(jnp.bfloat16),
      b1.reshape(1, -1), w2.astype(jnp.bfloat16), b2.reshape(1, -1),
      ss[0], ss[1])


def _sc_gather_body(tbl, send3d, recv3d, s_out, r_out,
                    sidx, ridx, bs0, bs1, br0, br1,
                    sem_s0, sem_s1, sem_r0, sem_r1):
    wid = lax.axis_index("s") * NC + lax.axis_index("c")
    pltpu.sync_copy(send3d.at[wid], sidx)
    pltpu.sync_copy(recv3d.at[wid], ridx)
    base = wid * EPW

    def issue(j, bs, br, ss, sr):
        pltpu.async_copy(tbl.at[sidx.at[j]], bs, ss)
        pltpu.async_copy(tbl.at[ridx.at[j]], br, sr)

    def drain(j, bs, br, ss, sr):
        pltpu.make_async_copy(tbl.at[sidx.at[j]], bs, ss).wait()
        pltpu.sync_copy(bs, s_out.at[pl.ds(base + j * GN, GN)])
        pltpu.make_async_copy(tbl.at[ridx.at[j]], br, sr).wait()
        pltpu.sync_copy(br, r_out.at[pl.ds(base + j * GN, GN)])

    issue(0, bs0, br0, sem_s0, sem_r0)

    def body(i, carry):
        j0 = 2 * i
        issue(j0 + 1, bs1, br1, sem_s1, sem_r1)
        drain(j0, bs0, br0, sem_s0, sem_r0)
        issue(j0 + 2, bs0, br0, sem_s0, sem_r0)
        drain(j0 + 1, bs1, br1, sem_s1, sem_r1)
        return carry

    lax.fori_loop(0, (NCHUNK - 1) // 2, body, 0)
    drain(NCHUNK - 1, bs0, br0, sem_s0, sem_r0)


@functools.cache
def _sc_gather():
    return pl.kernel(
        _sc_gather_body,
        mesh=_sc_mesh(),
        out_type=[jax.ShapeDtypeStruct((N_EDGES, 128), jnp.float32),
                  jax.ShapeDtypeStruct((N_EDGES, 128), jnp.float32)],
        scratch_types=[pltpu.VMEM((NCHUNK, GN), jnp.int32),
                       pltpu.VMEM((NCHUNK, GN), jnp.int32),
                       pltpu.VMEM((GN, 128), jnp.float32),
                       pltpu.VMEM((GN, 128), jnp.float32),
                       pltpu.VMEM((GN, 128), jnp.float32),
                       pltpu.VMEM((GN, 128), jnp.float32),
                       pltpu.SemaphoreType.DMA,
                       pltpu.SemaphoreType.DMA,
                       pltpu.SemaphoreType.DMA,
                       pltpu.SemaphoreType.DMA],
    )


def _sc_scatter_body(edge_hbm, recv3d, zeros128,
                     sums_out, idx_v, rows_v, st_v, acc_sh):
    cid = lax.axis_index("c")
    sid = lax.axis_index("s")
    wid = sid * NC + cid
    r0 = sid * ROWS_PER_TILE
    pltpu.sync_copy(zeros128, st_v)

    def zbody(k, carry):
        pltpu.sync_copy(st_v, acc_sh.at[pl.ds(r0 + k * ZCH, ZCH)])
        return carry

    lax.fori_loop(0, NZ, zbody, 0)
    pltpu.sync_copy(recv3d.at[wid], idx_v)
    plsc.subcore_barrier()

    def body(j, carry):
        pltpu.sync_copy(edge_hbm.at[pl.ds(wid * EPW + j * GN, GN)], rows_v)
        pltpu.sync_copy(rows_v, acc_sh.at[idx_v.at[j]], add=True)
        return carry

    lax.fori_loop(0, NCHUNK, body, 0)
    plsc.subcore_barrier()

    def obody(k, carry):
        pltpu.sync_copy(acc_sh.at[pl.ds(r0 + k * ZCH, ZCH)], st_v)
        pltpu.sync_copy(st_v, sums_out.at[cid, pl.ds(r0 + k * ZCH, ZCH)])
        return carry

    lax.fori_loop(0, NZ, obody, 0)


@functools.cache
def _sc_scatter():
    return pl.kernel(
        _sc_scatter_body,
        mesh=_sc_mesh(),
        out_type=jax.ShapeDtypeStruct((NC, NPAD, 128), jnp.float32),
        scratch_types=[pltpu.VMEM((NCHUNK, GN), jnp.int32),
                       pltpu.VMEM((GN, 128), jnp.float32),
                       pltpu.VMEM((ZCH, 128), jnp.float32),
                       pltpu.VMEM_SHARED((NPAD, 128), jnp.float32)],
    )


def _sc_counts_body(recv3d, zerosc, onesc,
                    cnts_out, idx_v, ones_v, stc_v, cnt_sh):
    cid = lax.axis_index("c")
    sid = lax.axis_index("s")
    wid = sid * NC + cid
    r0 = sid * ROWS_PER_TILE
    pltpu.sync_copy(zerosc, stc_v)

    def zbody(k, carry):
        pltpu.sync_copy(stc_v, cnt_sh.at[pl.ds(r0 + k * ZCH, ZCH)])
        return carry

    lax.fori_loop(0, NZ, zbody, 0)
    pltpu.sync_copy(onesc, ones_v)
    pltpu.sync_copy(recv3d.at[wid], idx_v)
    plsc.subcore_barrier()

    def body(j, carry):
        pltpu.sync_copy(ones_v, cnt_sh.at[idx_v.at[j]], add=True)
        return carry

    lax.fori_loop(0, NCHUNK, body, 0)
    plsc.subcore_barrier()

    def obody(k, carry):
        pltpu.sync_copy(cnt_sh.at[pl.ds(r0 + k * ZCH, ZCH)], stc_v)
        pltpu.sync_copy(stc_v, cnts_out.at[cid, pl.ds(r0 + k * ZCH, ZCH)])
        return carry

    lax.fori_loop(0, NZ, obody, 0)


@functools.cache
def _sc_counts():
    return pl.kernel(
        _sc_counts_body,
        mesh=_sc_mesh(),
        out_type=jax.ShapeDtypeStruct((NC, NPAD, CW), jnp.float32),
        scratch_types=[pltpu.VMEM((NCHUNK, GN), jnp.int32),
                       pltpu.VMEM((GN, CW), jnp.float32),
                       pltpu.VMEM((ZCH, CW), jnp.float32),
                       pltpu.VMEM_SHARED((NPAD, CW), jnp.float32)],
    )


def kernel(rnode_features, edge_features, senders, receivers, tau, params):
    tau11 = tau.astype(jnp.float32).reshape(1, 1)
    node_lat = rnode_features[:, 0, :]
    send3d = senders.reshape(NW, NCHUNK, GN)
    recv3d = receivers.reshape(NW, NCHUNK, GN)

    zeros128 = jnp.zeros((ZCH, 128), jnp.float32)
    zerosc = jnp.zeros((ZCH, CW), jnp.float32)
    onesc = jnp.ones((GN, CW), jnp.float32)

    lns = [params["embed_ln"]]
    for sp in params["steps"]:
        lns += [sp["edge_ln"], sp["node_ln"]]
    ss = _tc_tau(tau11, lns)

    edge_lat = _tc_embed(edge_features.astype(jnp.float32),
                         params["embed_mlp"], ss[0])
    cnts = _sc_counts()(recv3d, zerosc, onesc)

    for i, sp in enumerate(params["steps"]):
        s_buf, r_buf = _sc_gather()(node_lat, send3d, recv3d)
        edge_lat = _tc_edge(edge_lat, s_buf, r_buf,
                            sp["edge_mlp"], ss[1 + 2 * i])
        sums = _sc_scatter()(edge_lat, recv3d, zeros128)
        node_lat = _tc_node(node_lat, sums, cnts,
                            sp["node_mlp"], ss[2 + 2 * i])

    return node_lat[:, None, :]


# speedup vs baseline: 1.1442x; 1.0525x over previous
import functools

import jax
import jax.numpy as jnp
from jax import lax
from jax.experimental import pallas as pl
from jax.experimental.pallas import tpu as pltpu
from jax.experimental.pallas import tpu_sc as plsc

N_NODES = 10000
N_EDGES = 320000
NODE_LAT = 128
EDGE_LAT = 128

NC = 2
NS = 16
NW = NC * NS
EPW = N_EDGES // NW
GN = 80
NCHUNK = EPW // GN
NPAD = 10240
ROWS_PER_TILE = NPAD // NS
ZCH = 64
NZ = ROWS_PER_TILE // ZCH
CW = 128

BE = 6400
BN = 2000

@functools.cache
def _sc_mesh():
    return plsc.VectorSubcoreMesh(
        core_axis_name="c", subcore_axis_name="s",
        num_cores=NC, num_subcores=NS)


def _swish(x):
    return x * lax.logistic(x)


def _swish_bf(x):
    xb = x.astype(jnp.bfloat16)
    return xb * lax.logistic(xb)


def _ln_apply(x, scale, shift):
    ones = jnp.ones((x.shape[1], 1), jnp.float32)
    s1 = jnp.dot(x, ones, preferred_element_type=jnp.float32)
    s2 = jnp.dot(x * x, ones, preferred_element_type=jnp.float32)
    inv_d = 1.0 / x.shape[1]
    mu = s1 * inv_d
    var = s2 * inv_d - mu * mu
    inv = lax.rsqrt(var + 1e-5)
    return (x - mu) * inv * (1.0 + scale) + shift


def _tau_body(*refs):
    tau = refs[0][...]
    n_ln = (len(refs) - 1) // 8
    for i in range(n_ln):
        wc, bc, ws, bs, wb, bb = refs[1 + 6 * i:1 + 6 * i + 6]
        h = _swish(tau * wc[...] + bc[...])
        refs[1 + 6 * n_ln + 2 * i][...] = (
            jnp.dot(h, ws[...], preferred_element_type=jnp.float32) + bs[...])
        refs[1 + 6 * n_ln + 2 * i + 1][...] = (
            jnp.dot(h, wb[...], preferred_element_type=jnp.float32) + bb[...])


def _embed_body(ef_ref, w1_ref, b1_ref, w2_ref, b2_ref, sc_ref, sh_ref,
                out_ref):
    ef = ef_ref[...]
    w1 = w1_ref[...]
    h = b1_ref[...]
    for i in range(ef.shape[1]):
        h = h + ef[:, i:i + 1] * w1[i:i + 1, :]
    y = jnp.dot(_swish_bf(h), w2_ref[...],
                preferred_element_type=jnp.float32) + b2_ref[...]
    out_ref[...] = _ln_apply(y, sc_ref[...], sh_ref[...])


def _edge_body(el_ref, s_ref, r_ref, w1e_ref, w1s_ref, w1r_ref,
               b1_ref, w2_ref, b2_ref, sc_ref, sh_ref, out_ref):
    el = el_ref[...]
    h = (jnp.dot(el.astype(jnp.bfloat16), w1e_ref[...],
                 preferred_element_type=jnp.float32)
         + jnp.dot(s_ref[...].astype(jnp.bfloat16), w1s_ref[...],
                   preferred_element_type=jnp.float32)
         + jnp.dot(r_ref[...].astype(jnp.bfloat16), w1r_ref[...],
                   preferred_element_type=jnp.float32)
         + b1_ref[...])
    y = jnp.dot(_swish_bf(h), w2_ref[...],
                preferred_element_type=jnp.float32) + b2_ref[...]
    out_ref[...] = el + _ln_apply(y, sc_ref[...], sh_ref[...])


def _node_body(nl_ref, sum0_ref, sum1_ref, cnt0_ref, cnt1_ref,
               w1n_ref, w1a_ref, b1_ref, w2_ref, b2_ref, sc_ref, sh_ref,
               out_ref):
    nl = nl_ref[...]
    counts = cnt0_ref[:, 0:1] + cnt1_ref[:, 0:1]
    counts = jnp.maximum(counts, 1.0)
    agg = (sum0_ref[...] + sum1_ref[...]) / counts
    h = (jnp.dot(nl.astype(jnp.bfloat16), w1n_ref[...],
                 preferred_element_type=jnp.float32)
         + jnp.dot(agg.astype(jnp.bfloat16), w1a_ref[...],
                   preferred_element_type=jnp.float32)
         + b1_ref[...])
    y = jnp.dot(_swish_bf(h), w2_ref[...],
                preferred_element_type=jnp.float32) + b2_ref[...]
    out_ref[...] = nl + _ln_apply(y, sc_ref[...], sh_ref[...])


def _full(shape):
    return pl.BlockSpec(shape, lambda i: (0,) * len(shape))


def _rows(block_rows, cols):
    return pl.BlockSpec((block_rows, cols), lambda i: (i, 0))


def _tc_tau(tau11, lns):
    args = [tau11]
    for p in lns:
        args += [p["Wc"].reshape(1, -1), p["bc"].reshape(1, -1), p["Ws"],
                 p["bs"].reshape(1, -1), p["Wb"], p["bb"].reshape(1, -1)]
    outs = pl.pallas_call(
        _tau_body,
        out_shape=[jax.ShapeDtypeStruct((1, 128), jnp.float32)
                   for _ in range(2 * len(lns))],
    )(*args)
    return [(outs[2 * i], outs[2 * i + 1]) for i in range(len(lns))]


def _tc_embed(ef, mlp, ss):
    (w1, b1), (w2, b2) = mlp
    grid = (N_EDGES // BE,)
    return pl.pallas_call(
        _embed_body,
        grid=grid,
        in_specs=[_rows(BE, 4), _full((4, 256)), _full((1, 256)),
                  _full((256, 128)), _full((1, 128)),
                  _full((1, 128)), _full((1, 128))],
        out_specs=_rows(BE, 128),
        out_shape=jax.ShapeDtypeStruct((N_EDGES, 128), jnp.float32),
    )(ef, w1, b1.reshape(1, -1), w2.astype(jnp.bfloat16),
      b2.reshape(1, -1), ss[0], ss[1])


def _tc_edge(el, s, r, mlp, ss):
    (w1, b1), (w2, b2) = mlp
    grid = (N_EDGES // BE,)
    return pl.pallas_call(
        _edge_body,
        grid=grid,
        in_specs=[_rows(BE, 128), _rows(BE, 128), _rows(BE, 128),
                  _full((128, 256)), _full((128, 256)), _full((128, 256)),
                  _full((1, 256)), _full((256, 128)), _full((1, 128)),
                  _full((1, 128)), _full((1, 128))],
        out_specs=_rows(BE, 128),
        out_shape=jax.ShapeDtypeStruct((N_EDGES, 128), jnp.float32),
    )(el, s, r, w1[:128].astype(jnp.bfloat16),
      w1[128:256].astype(jnp.bfloat16), w1[256:].astype(jnp.bfloat16),
      b1.reshape(1, -1), w2.astype(jnp.bfloat16), b2.reshape(1, -1),
      ss[0], ss[1])


def _tc_node(nl, sums, cnts, mlp, ss):
    (w1, b1), (w2, b2) = mlp
    grid = (N_NODES // BN,)
    return pl.pallas_call(
        _node_body,
        grid=grid,
        in_specs=[_rows(BN, 128), _rows(BN, 128), _rows(BN, 128),
                  _rows(BN, CW), _rows(BN, CW),
                  _full((128, 256)), _full((128, 256)), _full((1, 256)),
                  _full((256, 128)), _full((1, 128)),
                  _full((1, 128)), _full((1, 128))],
        out_specs=_rows(BN, 128),
        out_shape=jax.ShapeDtypeStruct((N_NODES, 128), jnp.float32),
    )(nl, sums[0], sums[1], cnts[0], cnts[1],
      w1[:128].astype(jnp.bfloat16), w1[128:].astype(jnp.bfloat16),
      b1.reshape(1, -1), w2.astype(jnp.bfloat16), b2.reshape(1, -1),
      ss[0], ss[1])


def _sc_gather_body(tbl, send3d, recv3d, s_out, r_out,
                    sidx, ridx, bs0, bs1, br0, br1,
                    sem_s0, sem_s1, sem_r0, sem_r1):
    wid = lax.axis_index("s") * NC + lax.axis_index("c")
    pltpu.sync_copy(send3d.at[wid], sidx)
    pltpu.sync_copy(recv3d.at[wid], ridx)
    base = wid * EPW

    def issue(j, bs, br, ss, sr):
        pltpu.async_copy(tbl.at[sidx.at[j]], bs, ss)
        pltpu.async_copy(tbl.at[ridx.at[j]], br, sr)

    def drain(j, bs, br, ss, sr):
        pltpu.make_async_copy(tbl.at[sidx.at[j]], bs, ss).wait()
        pltpu.sync_copy(bs, s_out.at[pl.ds(base + j * GN, GN)])
        pltpu.make_async_copy(tbl.at[ridx.at[j]], br, sr).wait()
        pltpu.sync_copy(br, r_out.at[pl.ds(base + j * GN, GN)])

    issue(0, bs0, br0, sem_s0, sem_r0)

    def body(i, carry):
        j0 = 2 * i
        issue(j0 + 1, bs1, br1, sem_s1, sem_r1)
        drain(j0, bs0, br0, sem_s0, sem_r0)
        issue(j0 + 2, bs0, br0, sem_s0, sem_r0)
        drain(j0 + 1, bs1, br1, sem_s1, sem_r1)
        return carry

    lax.fori_loop(0, (NCHUNK - 1) // 2, body, 0)
    drain(NCHUNK - 1, bs0, br0, sem_s0, sem_r0)


@functools.cache
def _sc_gather():
    return pl.kernel(
        _sc_gather_body,
        mesh=_sc_mesh(),
        out_type=[jax.ShapeDtypeStruct((N_EDGES, 128), jnp.float32),
                  jax.ShapeDtypeStruct((N_EDGES, 128), jnp.float32)],
        scratch_types=[pltpu.VMEM((NCHUNK, GN), jnp.int32),
                       pltpu.VMEM((NCHUNK, GN), jnp.int32),
                       pltpu.VMEM((GN, 128), jnp.float32),
                       pltpu.VMEM((GN, 128), jnp.float32),
                       pltpu.VMEM((GN, 128), jnp.float32),
                       pltpu.VMEM((GN, 128), jnp.float32),
                       pltpu.SemaphoreType.DMA,
                       pltpu.SemaphoreType.DMA,
                       pltpu.SemaphoreType.DMA,
                       pltpu.SemaphoreType.DMA],
    )


def _sc_scatter_body(edge_hbm, recv3d, zeros128,
                     sums_out, idx_v, rows_v, st_v, acc_sh):
    cid = lax.axis_index("c")
    sid = lax.axis_index("s")
    wid = sid * NC + cid
    r0 = sid * ROWS_PER_TILE
    pltpu.sync_copy(zeros128, st_v)

    def zbody(k, carry):
        pltpu.sync_copy(st_v, acc_sh.at[pl.ds(r0 + k * ZCH, ZCH)])
        return carry

    lax.fori_loop(0, NZ, zbody, 0)
    pltpu.sync_copy(recv3d.at[wid], idx_v)
    plsc.subcore_barrier()

    def body(j, carry):
        pltpu.sync_copy(edge_hbm.at[pl.ds(wid * EPW + j * GN, GN)], rows_v)
        pltpu.sync_copy(rows_v, acc_sh.at[idx_v.at[j]], add=True)
        return carry

    lax.fori_loop(0, NCHUNK, body, 0)
    plsc.subcore_barrier()

    def obody(k, carry):
        pltpu.sync_copy(acc_sh.at[pl.ds(r0 + k * ZCH, ZCH)], st_v)
        pltpu.sync_copy(st_v, sums_out.at[cid, pl.ds(r0 + k * ZCH, ZCH)])
        return carry

    lax.fori_loop(0, NZ, obody, 0)


@functools.cache
def _sc_scatter():
    return pl.kernel(
        _sc_scatter_body,
        mesh=_sc_mesh(),
        out_type=jax.ShapeDtypeStruct((NC, NPAD, 128), jnp.float32),
        scratch_types=[pltpu.VMEM((NCHUNK, GN), jnp.int32),
                       pltpu.VMEM((GN, 128), jnp.float32),
                       pltpu.VMEM((ZCH, 128), jnp.float32),
                       pltpu.VMEM_SHARED((NPAD, 128), jnp.float32)],
    )


def _sc_counts_body(recv3d, zerosc, onesc,
                    cnts_out, idx_v, ones_v, stc_v, cnt_sh):
    cid = lax.axis_index("c")
    sid = lax.axis_index("s")
    wid = sid * NC + cid
    r0 = sid * ROWS_PER_TILE
    pltpu.sync_copy(zerosc, stc_v)

    def zbody(k, carry):
        pltpu.sync_copy(stc_v, cnt_sh.at[pl.ds(r0 + k * ZCH, ZCH)])
        return carry

    lax.fori_loop(0, NZ, zbody, 0)
    pltpu.sync_copy(onesc, ones_v)
    pltpu.sync_copy(recv3d.at[wid], idx_v)
    plsc.subcore_barrier()

    def body(j, carry):
        pltpu.sync_copy(ones_v, cnt_sh.at[idx_v.at[j]], add=True)
        return carry

    lax.fori_loop(0, NCHUNK, body, 0)
    plsc.subcore_barrier()

    def obody(k, carry):
        pltpu.sync_copy(cnt_sh.at[pl.ds(r0 + k * ZCH, ZCH)], stc_v)
        pltpu.sync_copy(stc_v, cnts_out.at[cid, pl.ds(r0 + k * ZCH, ZCH)])
        return carry

    lax.fori_loop(0, NZ, obody, 0)


@functools.cache
def _sc_counts():
    return pl.kernel(
        _sc_counts_body,
        mesh=_sc_mesh(),
        out_type=jax.ShapeDtypeStruct((NC, NPAD, CW), jnp.float32),
        scratch_types=[pltpu.VMEM((NCHUNK, GN), jnp.int32),
                       pltpu.VMEM((GN, CW), jnp.float32),
                       pltpu.VMEM((ZCH, CW), jnp.float32),
                       pltpu.VMEM_SHARED((NPAD, CW), jnp.float32)],
    )


def kernel(rnode_features, edge_features, senders, receivers, tau, params):
    tau11 = tau.astype(jnp.float32).reshape(1, 1)
    node_lat = rnode_features[:, 0, :]
    send3d = senders.reshape(NW, NCHUNK, GN)
    recv3d = receivers.reshape(NW, NCHUNK, GN)

    zeros128 = jnp.zeros((ZCH, 128), jnp.float32)
    zerosc = jnp.zeros((ZCH, CW), jnp.float32)
    onesc = jnp.ones((GN, CW), jnp.float32)

    lns = [params["embed_ln"]]
    for sp in params["steps"]:
        lns += [sp["edge_ln"], sp["node_ln"]]
    ss = _tc_tau(tau11, lns)

    edge_lat = _tc_embed(edge_features.astype(jnp.float32),
                         params["embed_mlp"], ss[0])
    cnts = _sc_counts()(recv3d, zerosc, onesc)

    for i, sp in enumerate(params["steps"]):
        s_buf, r_buf = _sc_gather()(node_lat, send3d, recv3d)
        edge_lat = _tc_edge(edge_lat, s_buf, r_buf,
                            sp["edge_mlp"], ss[1 + 2 * i])
        sums = _sc_scatter()(edge_lat, recv3d, zeros128)
        node_lat = _tc_node(node_lat, sums, cnts,
                            sp["node_mlp"], ss[2 + 2 * i])

    return node_lat[:, None, :]


# speedup vs baseline: 1.1474x; 1.0027x over previous
import functools

import jax
import jax.numpy as jnp
from jax import lax
from jax.experimental import pallas as pl
from jax.experimental.pallas import tpu as pltpu
from jax.experimental.pallas import tpu_sc as plsc

N_NODES = 10000
N_EDGES = 320000
NODE_LAT = 128
EDGE_LAT = 128

NC = 2
NS = 16
NW = NC * NS
EPW = N_EDGES // NW
GN = 80
NCHUNK = EPW // GN
NPAD = 10240
ROWS_PER_TILE = NPAD // NS
ZCH = 64
NZ = ROWS_PER_TILE // ZCH
CW = 128

BE = 8000
BN = 2000

@functools.cache
def _sc_mesh():
    return plsc.VectorSubcoreMesh(
        core_axis_name="c", subcore_axis_name="s",
        num_cores=NC, num_subcores=NS)


def _swish(x):
    return x * lax.logistic(x)


def _swish_bf(x):
    xb = x.astype(jnp.bfloat16)
    return xb * lax.logistic(xb)


def _ln_apply(x, scale, shift):
    ones = jnp.ones((x.shape[1], 1), jnp.float32)
    s1 = jnp.dot(x, ones, preferred_element_type=jnp.float32)
    s2 = jnp.dot(x * x, ones, preferred_element_type=jnp.float32)
    inv_d = 1.0 / x.shape[1]
    mu = s1 * inv_d
    var = s2 * inv_d - mu * mu
    inv = lax.rsqrt(var + 1e-5)
    return (x - mu) * inv * (1.0 + scale) + shift


def _tau_body(*refs):
    tau = refs[0][...]
    n_ln = (len(refs) - 1) // 8
    for i in range(n_ln):
        wc, bc, ws, bs, wb, bb = refs[1 + 6 * i:1 + 6 * i + 6]
        h = _swish(tau * wc[...] + bc[...])
        refs[1 + 6 * n_ln + 2 * i][...] = (
            jnp.dot(h, ws[...], preferred_element_type=jnp.float32) + bs[...])
        refs[1 + 6 * n_ln + 2 * i + 1][...] = (
            jnp.dot(h, wb[...], preferred_element_type=jnp.float32) + bb[...])


def _embed_body(ef_ref, w1_ref, b1_ref, w2_ref, b2_ref, sc_ref, sh_ref,
                out_ref):
    ef = ef_ref[...]
    w1 = w1_ref[...]
    h = b1_ref[...]
    for i in range(ef.shape[1]):
        h = h + ef[:, i:i + 1] * w1[i:i + 1, :]
    y = jnp.dot(_swish_bf(h), w2_ref[...],
                preferred_element_type=jnp.float32) + b2_ref[...]
    out_ref[...] = _ln_apply(y, sc_ref[...], sh_ref[...])


def _edge_body(el_ref, s_ref, r_ref, w1e_ref, w1s_ref, w1r_ref,
               b1_ref, w2_ref, b2_ref, sc_ref, sh_ref, out_ref):
    el = el_ref[...]
    h = (jnp.dot(el.astype(jnp.bfloat16), w1e_ref[...],
                 preferred_element_type=jnp.float32)
         + jnp.dot(s_ref[...].astype(jnp.bfloat16), w1s_ref[...],
                   preferred_element_type=jnp.float32)
         + jnp.dot(r_ref[...].astype(jnp.bfloat16), w1r_ref[...],
                   preferred_element_type=jnp.float32)
         + b1_ref[...])
    y = jnp.dot(_swish_bf(h), w2_ref[...],
                preferred_element_type=jnp.float32) + b2_ref[...]
    out_ref[...] = el + _ln_apply(y, sc_ref[...], sh_ref[...])


def _node_body(nl_ref, sum0_ref, sum1_ref, cnt0_ref, cnt1_ref,
               w1n_ref, w1a_ref, b1_ref, w2_ref, b2_ref, sc_ref, sh_ref,
               out_ref):
    nl = nl_ref[...]
    counts = cnt0_ref[:, 0:1] + cnt1_ref[:, 0:1]
    counts = jnp.maximum(counts, 1.0)
    agg = (sum0_ref[...] + sum1_ref[...]) / counts
    h = (jnp.dot(nl.astype(jnp.bfloat16), w1n_ref[...],
                 preferred_element_type=jnp.float32)
         + jnp.dot(agg.astype(jnp.bfloat16), w1a_ref[...],
                   preferred_element_type=jnp.float32)
         + b1_ref[...])
    y = jnp.dot(_swish_bf(h), w2_ref[...],
                preferred_element_type=jnp.float32) + b2_ref[...]
    out_ref[...] = nl + _ln_apply(y, sc_ref[...], sh_ref[...])


def _full(shape):
    return pl.BlockSpec(shape, lambda i: (0,) * len(shape))


def _rows(block_rows, cols):
    return pl.BlockSpec((block_rows, cols), lambda i: (i, 0))


def _tc_tau(tau11, lns):
    args = [tau11]
    for p in lns:
        args += [p["Wc"].reshape(1, -1), p["bc"].reshape(1, -1), p["Ws"],
                 p["bs"].reshape(1, -1), p["Wb"], p["bb"].reshape(1, -1)]
    outs = pl.pallas_call(
        _tau_body,
        out_shape=[jax.ShapeDtypeStruct((1, 128), jnp.float32)
                   for _ in range(2 * len(lns))],
    )(*args)
    return [(outs[2 * i], outs[2 * i + 1]) for i in range(len(lns))]


def _tc_embed(ef, mlp, ss):
    (w1, b1), (w2, b2) = mlp
    grid = (N_EDGES // BE,)
    return pl.pallas_call(
        _embed_body,
        grid=grid,
        in_specs=[_rows(BE, 4), _full((4, 256)), _full((1, 256)),
                  _full((256, 128)), _full((1, 128)),
                  _full((1, 128)), _full((1, 128))],
        out_specs=_rows(BE, 128),
        out_shape=jax.ShapeDtypeStruct((N_EDGES, 128), jnp.float32),
    )(ef, w1, b1.reshape(1, -1), w2.astype(jnp.bfloat16),
      b2.reshape(1, -1), ss[0], ss[1])


def _tc_edge(el, s, r, mlp, ss):
    (w1, b1), (w2, b2) = mlp
    grid = (N_EDGES // BE,)
    return pl.pallas_call(
        _edge_body,
        grid=grid,
        in_specs=[_rows(BE, 128), _rows(BE, 128), _rows(BE, 128),
                  _full((128, 256)), _full((128, 256)), _full((128, 256)),
                  _full((1, 256)), _full((256, 128)), _full((1, 128)),
                  _full((1, 128)), _full((1, 128))],
        out_specs=_rows(BE, 128),
        out_shape=jax.ShapeDtypeStruct((N_EDGES, 128), jnp.float32),
    )(el, s, r, w1[:128].astype(jnp.bfloat16),
      w1[128:256].astype(jnp.bfloat16), w1[256:].astype(jnp.bfloat16),
      b1.reshape(1, -1), w2.astype(jnp.bfloat16), b2.reshape(1, -1),
      ss[0], ss[1])


def _tc_node(nl, sums, cnts, mlp, ss):
    (w1, b1), (w2, b2) = mlp
    grid = (N_NODES // BN,)
    return pl.pallas_call(
        _node_body,
        grid=grid,
        in_specs=[_rows(BN, 128), _rows(BN, 128), _rows(BN, 128),
                  _rows(BN, CW), _rows(BN, CW),
                  _full((128, 256)), _full((128, 256)), _full((1, 256)),
                  _full((256, 128)), _full((1, 128)),
                  _full((1, 128)), _full((1, 128))],
        out_specs=_rows(BN, 128),
        out_shape=jax.ShapeDtypeStruct((N_NODES, 128), jnp.float32),
    )(nl, sums[0], sums[1], cnts[0], cnts[1],
      w1[:128].astype(jnp.bfloat16), w1[128:].astype(jnp.bfloat16),
      b1.reshape(1, -1), w2.astype(jnp.bfloat16), b2.reshape(1, -1),
      ss[0], ss[1])


def _sc_gather_body(tbl, send3d, recv3d, s_out, r_out,
                    sidx, ridx, bs0, bs1, br0, br1,
                    sem_s0, sem_s1, sem_r0, sem_r1):
    wid = lax.axis_index("s") * NC + lax.axis_index("c")
    pltpu.sync_copy(send3d.at[wid], sidx)
    pltpu.sync_copy(recv3d.at[wid], ridx)
    base = wid * EPW

    def issue(j, bs, br, ss, sr):
        pltpu.async_copy(tbl.at[sidx.at[j]], bs, ss)
        pltpu.async_copy(tbl.at[ridx.at[j]], br, sr)

    def drain(j, bs, br, ss, sr):
        pltpu.make_async_copy(tbl.at[sidx.at[j]], bs, ss).wait()
        pltpu.sync_copy(bs, s_out.at[pl.ds(base + j * GN, GN)])
        pltpu.make_async_copy(tbl.at[ridx.at[j]], br, sr).wait()
        pltpu.sync_copy(br, r_out.at[pl.ds(base + j * GN, GN)])

    issue(0, bs0, br0, sem_s0, sem_r0)

    def body(i, carry):
        j0 = 2 * i
        issue(j0 + 1, bs1, br1, sem_s1, sem_r1)
        drain(j0, bs0, br0, sem_s0, sem_r0)
        issue(j0 + 2, bs0, br0, sem_s0, sem_r0)
        drain(j0 + 1, bs1, br1, sem_s1, sem_r1)
        return carry

    lax.fori_loop(0, (NCHUNK - 1) // 2, body, 0)
    drain(NCHUNK - 1, bs0, br0, sem_s0, sem_r0)


@functools.cache
def _sc_gather():
    return pl.kernel(
        _sc_gather_body,
        mesh=_sc_mesh(),
        out_type=[jax.ShapeDtypeStruct((N_EDGES, 128), jnp.float32),
                  jax.ShapeDtypeStruct((N_EDGES, 128), jnp.float32)],
        scratch_types=[pltpu.VMEM((NCHUNK, GN), jnp.int32),
                       pltpu.VMEM((NCHUNK, GN), jnp.int32),
                       pltpu.VMEM((GN, 128), jnp.float32),
                       pltpu.VMEM((GN, 128), jnp.float32),
                       pltpu.VMEM((GN, 128), jnp.float32),
                       pltpu.VMEM((GN, 128), jnp.float32),
                       pltpu.SemaphoreType.DMA,
                       pltpu.SemaphoreType.DMA,
                       pltpu.SemaphoreType.DMA,
                       pltpu.SemaphoreType.DMA],
    )


def _sc_scatter_body(edge_hbm, recv3d, zeros128,
                     sums_out, idx_v, rows_v, st_v, acc_sh):
    cid = lax.axis_index("c")
    sid = lax.axis_index("s")
    wid = sid * NC + cid
    r0 = sid * ROWS_PER_TILE
    pltpu.sync_copy(zeros128, st_v)

    def zbody(k, carry):
        pltpu.sync_copy(st_v, acc_sh.at[pl.ds(r0 + k * ZCH, ZCH)])
        return carry

    lax.fori_loop(0, NZ, zbody, 0)
    pltpu.sync_copy(recv3d.at[wid], idx_v)
    plsc.subcore_barrier()

    def body(j, carry):
        pltpu.sync_copy(edge_hbm.at[pl.ds(wid * EPW + j * GN, GN)], rows_v)
        pltpu.sync_copy(rows_v, acc_sh.at[idx_v.at[j]], add=True)
        return carry

    lax.fori_loop(0, NCHUNK, body, 0)
    plsc.subcore_barrier()

    def obody(k, carry):
        pltpu.sync_copy(acc_sh.at[pl.ds(r0 + k * ZCH, ZCH)], st_v)
        pltpu.sync_copy(st_v, sums_out.at[cid, pl.ds(r0 + k * ZCH, ZCH)])
        return carry

    lax.fori_loop(0, NZ, obody, 0)


@functools.cache
def _sc_scatter():
    return pl.kernel(
        _sc_scatter_body,
        mesh=_sc_mesh(),
        out_type=jax.ShapeDtypeStruct((NC, NPAD, 128), jnp.float32),
        scratch_types=[pltpu.VMEM((NCHUNK, GN), jnp.int32),
                       pltpu.VMEM((GN, 128), jnp.float32),
                       pltpu.VMEM((ZCH, 128), jnp.float32),
                       pltpu.VMEM_SHARED((NPAD, 128), jnp.float32)],
    )


def _sc_counts_body(recv3d, zerosc, onesc,
                    cnts_out, idx_v, ones_v, stc_v, cnt_sh):
    cid = lax.axis_index("c")
    sid = lax.axis_index("s")
    wid = sid * NC + cid
    r0 = sid * ROWS_PER_TILE
    pltpu.sync_copy(zerosc, stc_v)

    def zbody(k, carry):
        pltpu.sync_copy(stc_v, cnt_sh.at[pl.ds(r0 + k * ZCH, ZCH)])
        return carry

    lax.fori_loop(0, NZ, zbody, 0)
    pltpu.sync_copy(onesc, ones_v)
    pltpu.sync_copy(recv3d.at[wid], idx_v)
    plsc.subcore_barrier()

    def body(j, carry):
        pltpu.sync_copy(ones_v, cnt_sh.at[idx_v.at[j]], add=True)
        return carry

    lax.fori_loop(0, NCHUNK, body, 0)
    plsc.subcore_barrier()

    def obody(k, carry):
        pltpu.sync_copy(cnt_sh.at[pl.ds(r0 + k * ZCH, ZCH)], stc_v)
        pltpu.sync_copy(stc_v, cnts_out.at[cid, pl.ds(r0 + k * ZCH, ZCH)])
        return carry

    lax.fori_loop(0, NZ, obody, 0)


@functools.cache
def _sc_counts():
    return pl.kernel(
        _sc_counts_body,
        mesh=_sc_mesh(),
        out_type=jax.ShapeDtypeStruct((NC, NPAD, CW), jnp.float32),
        scratch_types=[pltpu.VMEM((NCHUNK, GN), jnp.int32),
                       pltpu.VMEM((GN, CW), jnp.float32),
                       pltpu.VMEM((ZCH, CW), jnp.float32),
                       pltpu.VMEM_SHARED((NPAD, CW), jnp.float32)],
    )


def kernel(rnode_features, edge_features, senders, receivers, tau, params):
    tau11 = tau.astype(jnp.float32).reshape(1, 1)
    node_lat = rnode_features[:, 0, :]
    send3d = senders.reshape(NW, NCHUNK, GN)
    recv3d = receivers.reshape(NW, NCHUNK, GN)

    zeros128 = jnp.zeros((ZCH, 128), jnp.float32)
    zerosc = jnp.zeros((ZCH, CW), jnp.float32)
    onesc = jnp.ones((GN, CW), jnp.float32)

    lns = [params["embed_ln"]]
    for sp in params["steps"]:
        lns += [sp["edge_ln"], sp["node_ln"]]
    ss = _tc_tau(tau11, lns)

    edge_lat = _tc_embed(edge_features.astype(jnp.float32),
                         params["embed_mlp"], ss[0])
    cnts = _sc_counts()(recv3d, zerosc, onesc)

    for i, sp in enumerate(params["steps"]):
        s_buf, r_buf = _sc_gather()(node_lat, send3d, recv3d)
        edge_lat = _tc_edge(edge_lat, s_buf, r_buf,
                            sp["edge_mlp"], ss[1 + 2 * i])
        sums = _sc_scatter()(edge_lat, recv3d, zeros128)
        node_lat = _tc_node(node_lat, sums, cnts,
                            sp["node_mlp"], ss[2 + 2 * i])

    return node_lat[:, None, :]


# speedup vs baseline: 1.2606x; 1.0987x over previous
import functools

import jax
import jax.numpy as jnp
from jax import lax
from jax.experimental import pallas as pl
from jax.experimental.pallas import tpu as pltpu
from jax.experimental.pallas import tpu_sc as plsc

N_NODES = 10000
N_EDGES = 320000
NODE_LAT = 128
EDGE_LAT = 128

NC = 2
NS = 16
NW = NC * NS
EPW = N_EDGES // NW
GN = 80
NCHUNK = EPW // GN
NPAD = 10240
ROWS_PER_TILE = NPAD // NS
ZCH = 64
NZ = ROWS_PER_TILE // ZCH
CW = 128

BE = 8000
BN = 2000

@functools.cache
def _sc_mesh():
    return plsc.VectorSubcoreMesh(
        core_axis_name="c", subcore_axis_name="s",
        num_cores=NC, num_subcores=NS)


def _swish(x):
    return x * lax.logistic(x)


def _swish_bf(x):
    xb = x.astype(jnp.bfloat16)
    return xb * lax.logistic(xb)


def _ln_apply(x, scale, shift):
    ones = jnp.ones((x.shape[1], 1), jnp.float32)
    s1 = jnp.dot(x, ones, preferred_element_type=jnp.float32)
    s2 = jnp.dot(x * x, ones, preferred_element_type=jnp.float32)
    inv_d = 1.0 / x.shape[1]
    mu = s1 * inv_d
    var = s2 * inv_d - mu * mu
    inv = lax.rsqrt(var + 1e-5)
    return (x - mu) * inv * (1.0 + scale) + shift


def _tau_body(*refs):
    tau = refs[0][...]
    n_ln = (len(refs) - 1) // 8
    for i in range(n_ln):
        wc, bc, ws, bs, wb, bb = refs[1 + 6 * i:1 + 6 * i + 6]
        h = _swish(tau * wc[...] + bc[...])
        refs[1 + 6 * n_ln + 2 * i][...] = (
            jnp.dot(h, ws[...], preferred_element_type=jnp.float32) + bs[...])
        refs[1 + 6 * n_ln + 2 * i + 1][...] = (
            jnp.dot(h, wb[...], preferred_element_type=jnp.float32) + bb[...])


def _embed_body(ef_ref, w1_ref, b1_ref, w2_ref, b2_ref, sc_ref, sh_ref,
                out_ref):
    ef = ef_ref[...]
    w1 = w1_ref[...]
    h = b1_ref[...]
    for i in range(ef.shape[1]):
        h = h + ef[:, i:i + 1] * w1[i:i + 1, :]
    y = jnp.dot(_swish_bf(h), w2_ref[...],
                preferred_element_type=jnp.float32) + b2_ref[...]
    out_ref[...] = _ln_apply(y, sc_ref[...], sh_ref[...])


def _edge_body(el_ref, s_ref, r_ref, w1e_ref, w1s_ref, w1r_ref,
               b1_ref, w2_ref, b2_ref, sc_ref, sh_ref, out_ref):
    el = el_ref[...]
    h = (jnp.dot(el.astype(jnp.bfloat16), w1e_ref[...],
                 preferred_element_type=jnp.float32)
         + jnp.dot(s_ref[...].astype(jnp.bfloat16), w1s_ref[...],
                   preferred_element_type=jnp.float32)
         + jnp.dot(r_ref[...].astype(jnp.bfloat16), w1r_ref[...],
                   preferred_element_type=jnp.float32)
         + b1_ref[...])
    y = jnp.dot(_swish_bf(h), w2_ref[...],
                preferred_element_type=jnp.float32) + b2_ref[...]
    out_ref[...] = el + _ln_apply(y, sc_ref[...], sh_ref[...])


def _node_body(nl_ref, sum0_ref, sum1_ref, cnt0_ref, cnt1_ref,
               w1n_ref, w1a_ref, b1_ref, w2_ref, b2_ref, sc_ref, sh_ref,
               out_ref):
    nl = nl_ref[...]
    counts = cnt0_ref[:, 0:1] + cnt1_ref[:, 0:1]
    counts = jnp.maximum(counts, 1.0)
    agg = (sum0_ref[...] + sum1_ref[...]) / counts
    h = (jnp.dot(nl.astype(jnp.bfloat16), w1n_ref[...],
                 preferred_element_type=jnp.float32)
         + jnp.dot(agg.astype(jnp.bfloat16), w1a_ref[...],
                   preferred_element_type=jnp.float32)
         + b1_ref[...])
    y = jnp.dot(_swish_bf(h), w2_ref[...],
                preferred_element_type=jnp.float32) + b2_ref[...]
    out_ref[...] = nl + _ln_apply(y, sc_ref[...], sh_ref[...])


def _full(shape):
    return pl.BlockSpec(shape, lambda i: (0,) * len(shape))


def _rows(block_rows, cols):
    return pl.BlockSpec((block_rows, cols), lambda i: (i, 0))


def _tc_tau(tau11, lns):
    args = [tau11]
    for p in lns:
        args += [p["Wc"].reshape(1, -1), p["bc"].reshape(1, -1), p["Ws"],
                 p["bs"].reshape(1, -1), p["Wb"], p["bb"].reshape(1, -1)]
    outs = pl.pallas_call(
        _tau_body,
        out_shape=[jax.ShapeDtypeStruct((1, 128), jnp.float32)
                   for _ in range(2 * len(lns))],
    )(*args)
    return [(outs[2 * i], outs[2 * i + 1]) for i in range(len(lns))]


def _tc_embed(ef, mlp, ss):
    (w1, b1), (w2, b2) = mlp
    grid = (N_EDGES // BE,)
    return pl.pallas_call(
        _embed_body,
        grid=grid,
        in_specs=[_rows(BE, 4), _full((4, 256)), _full((1, 256)),
                  _full((256, 128)), _full((1, 128)),
                  _full((1, 128)), _full((1, 128))],
        out_specs=_rows(BE, 128),
        out_shape=jax.ShapeDtypeStruct((N_EDGES, 128), jnp.float32),
    )(ef, w1, b1.reshape(1, -1), w2.astype(jnp.bfloat16),
      b2.reshape(1, -1), ss[0], ss[1])


def _tc_edge(el, s, r, mlp, ss):
    (w1, b1), (w2, b2) = mlp
    grid = (N_EDGES // BE,)
    return pl.pallas_call(
        _edge_body,
        grid=grid,
        in_specs=[_rows(BE, 128), _rows(BE, 128), _rows(BE, 128),
                  _full((128, 256)), _full((128, 256)), _full((128, 256)),
                  _full((1, 256)), _full((256, 128)), _full((1, 128)),
                  _full((1, 128)), _full((1, 128))],
        out_specs=_rows(BE, 128),
        out_shape=jax.ShapeDtypeStruct((N_EDGES, 128), jnp.float32),
    )(el, s, r, w1[:128].astype(jnp.bfloat16),
      w1[128:256].astype(jnp.bfloat16), w1[256:].astype(jnp.bfloat16),
      b1.reshape(1, -1), w2.astype(jnp.bfloat16), b2.reshape(1, -1),
      ss[0], ss[1])


def _tc_node(nl, sums, cnts, mlp, ss):
    (w1, b1), (w2, b2) = mlp
    grid = (N_NODES // BN,)
    return pl.pallas_call(
        _node_body,
        grid=grid,
        in_specs=[_rows(BN, 128), _rows(BN, 128), _rows(BN, 128),
                  _rows(BN, CW), _rows(BN, CW),
                  _full((128, 256)), _full((128, 256)), _full((1, 256)),
                  _full((256, 128)), _full((1, 128)),
                  _full((1, 128)), _full((1, 128))],
        out_specs=_rows(BN, 128),
        out_shape=jax.ShapeDtypeStruct((N_NODES, 128), jnp.float32),
    )(nl, sums[0], sums[1], cnts[0], cnts[1],
      w1[:128].astype(jnp.bfloat16), w1[128:].astype(jnp.bfloat16),
      b1.reshape(1, -1), w2.astype(jnp.bfloat16), b2.reshape(1, -1),
      ss[0], ss[1])


def _sc_gather_body(tbl, send3d, recv3d, s_out, r_out,
                    sidx, ridx, bs0, bs1, br0, br1,
                    sem_s0, sem_s1, sem_r0, sem_r1):
    wid = lax.axis_index("s") * NC + lax.axis_index("c")
    pltpu.sync_copy(send3d.at[wid], sidx)
    pltpu.sync_copy(recv3d.at[wid], ridx)
    base = wid * EPW

    def issue(j, bs, br, ss, sr):
        pltpu.async_copy(tbl.at[sidx.at[j]], bs, ss)
        pltpu.async_copy(tbl.at[ridx.at[j]], br, sr)

    def drain(j, bs, br, ss, sr):
        pltpu.make_async_copy(tbl.at[sidx.at[j]], bs, ss).wait()
        pltpu.sync_copy(bs, s_out.at[pl.ds(base + j * GN, GN)])
        pltpu.make_async_copy(tbl.at[ridx.at[j]], br, sr).wait()
        pltpu.sync_copy(br, r_out.at[pl.ds(base + j * GN, GN)])

    issue(0, bs0, br0, sem_s0, sem_r0)

    def body(i, carry):
        j0 = 2 * i
        issue(j0 + 1, bs1, br1, sem_s1, sem_r1)
        drain(j0, bs0, br0, sem_s0, sem_r0)
        issue(j0 + 2, bs0, br0, sem_s0, sem_r0)
        drain(j0 + 1, bs1, br1, sem_s1, sem_r1)
        return carry

    lax.fori_loop(0, (NCHUNK - 1) // 2, body, 0)
    drain(NCHUNK - 1, bs0, br0, sem_s0, sem_r0)


@functools.cache
def _sc_gather():
    return pl.kernel(
        _sc_gather_body,
        mesh=_sc_mesh(),
        out_type=[jax.ShapeDtypeStruct((N_EDGES, 128), jnp.float32),
                  jax.ShapeDtypeStruct((N_EDGES, 128), jnp.float32)],
        scratch_types=[pltpu.VMEM((NCHUNK, GN), jnp.int32),
                       pltpu.VMEM((NCHUNK, GN), jnp.int32),
                       pltpu.VMEM((GN, 128), jnp.float32),
                       pltpu.VMEM((GN, 128), jnp.float32),
                       pltpu.VMEM((GN, 128), jnp.float32),
                       pltpu.VMEM((GN, 128), jnp.float32),
                       pltpu.SemaphoreType.DMA,
                       pltpu.SemaphoreType.DMA,
                       pltpu.SemaphoreType.DMA,
                       pltpu.SemaphoreType.DMA],
    )


def _sc_scatter_body(edge_hbm, recv3d, zeros128,
                     sums_out, idx_v, rows_v, rows_b, st_v, acc_sh,
                     sem_a, sem_b):
    cid = lax.axis_index("c")
    sid = lax.axis_index("s")
    wid = sid * NC + cid
    r0 = sid * ROWS_PER_TILE
    pltpu.sync_copy(zeros128, st_v)

    def zbody(k, carry):
        pltpu.sync_copy(st_v, acc_sh.at[pl.ds(r0 + k * ZCH, ZCH)])
        return carry

    lax.fori_loop(0, NZ, zbody, 0)
    pltpu.sync_copy(recv3d.at[wid], idx_v)
    plsc.subcore_barrier()
    base = wid * EPW

    def issue(j, buf, sem):
        pltpu.async_copy(edge_hbm.at[pl.ds(base + j * GN, GN)], buf, sem)

    def drain(j, buf, sem):
        pltpu.make_async_copy(
            edge_hbm.at[pl.ds(base + j * GN, GN)], buf, sem).wait()
        pltpu.sync_copy(buf, acc_sh.at[idx_v.at[j]], add=True)

    issue(0, rows_v, sem_a)

    def body(i, carry):
        j0 = 2 * i
        issue(j0 + 1, rows_b, sem_b)
        drain(j0, rows_v, sem_a)
        issue(j0 + 2, rows_v, sem_a)
        drain(j0 + 1, rows_b, sem_b)
        return carry

    lax.fori_loop(0, (NCHUNK - 1) // 2, body, 0)
    drain(NCHUNK - 1, rows_v, sem_a)
    plsc.subcore_barrier()

    def obody(k, carry):
        pltpu.sync_copy(acc_sh.at[pl.ds(r0 + k * ZCH, ZCH)], st_v)
        pltpu.sync_copy(st_v, sums_out.at[cid, pl.ds(r0 + k * ZCH, ZCH)])
        return carry

    lax.fori_loop(0, NZ, obody, 0)


@functools.cache
def _sc_scatter():
    return pl.kernel(
        _sc_scatter_body,
        mesh=_sc_mesh(),
        out_type=jax.ShapeDtypeStruct((NC, NPAD, 128), jnp.float32),
        scratch_types=[pltpu.VMEM((NCHUNK, GN), jnp.int32),
                       pltpu.VMEM((GN, 128), jnp.float32),
                       pltpu.VMEM((GN, 128), jnp.float32),
                       pltpu.VMEM((ZCH, 128), jnp.float32),
                       pltpu.VMEM_SHARED((NPAD, 128), jnp.float32),
                       pltpu.SemaphoreType.DMA,
                       pltpu.SemaphoreType.DMA],
    )


def _sc_counts_body(recv3d, zerosc, onesc,
                    cnts_out, idx_v, ones_v, stc_v, cnt_sh):
    cid = lax.axis_index("c")
    sid = lax.axis_index("s")
    wid = sid * NC + cid
    r0 = sid * ROWS_PER_TILE
    pltpu.sync_copy(zerosc, stc_v)

    def zbody(k, carry):
        pltpu.sync_copy(stc_v, cnt_sh.at[pl.ds(r0 + k * ZCH, ZCH)])
        return carry

    lax.fori_loop(0, NZ, zbody, 0)
    pltpu.sync_copy(onesc, ones_v)
    pltpu.sync_copy(recv3d.at[wid], idx_v)
    plsc.subcore_barrier()

    def body(j, carry):
        pltpu.sync_copy(ones_v, cnt_sh.at[idx_v.at[j]], add=True)
        return carry

    lax.fori_loop(0, NCHUNK, body, 0)
    plsc.subcore_barrier()

    def obody(k, carry):
        pltpu.sync_copy(cnt_sh.at[pl.ds(r0 + k * ZCH, ZCH)], stc_v)
        pltpu.sync_copy(stc_v, cnts_out.at[cid, pl.ds(r0 + k * ZCH, ZCH)])
        return carry

    lax.fori_loop(0, NZ, obody, 0)


@functools.cache
def _sc_counts():
    return pl.kernel(
        _sc_counts_body,
        mesh=_sc_mesh(),
        out_type=jax.ShapeDtypeStruct((NC, NPAD, CW), jnp.float32),
        scratch_types=[pltpu.VMEM((NCHUNK, GN), jnp.int32),
                       pltpu.VMEM((GN, CW), jnp.float32),
                       pltpu.VMEM((ZCH, CW), jnp.float32),
                       pltpu.VMEM_SHARED((NPAD, CW), jnp.float32)],
    )


def kernel(rnode_features, edge_features, senders, receivers, tau, params):
    tau11 = tau.astype(jnp.float32).reshape(1, 1)
    node_lat = rnode_features[:, 0, :]
    send3d = senders.reshape(NW, NCHUNK, GN)
    recv3d = receivers.reshape(NW, NCHUNK, GN)

    zeros128 = jnp.zeros((ZCH, 128), jnp.float32)
    zerosc = jnp.zeros((ZCH, CW), jnp.float32)
    onesc = jnp.ones((GN, CW), jnp.float32)

    lns = [params["embed_ln"]]
    for sp in params["steps"]:
        lns += [sp["edge_ln"], sp["node_ln"]]
    ss = _tc_tau(tau11, lns)

    edge_lat = _tc_embed(edge_features.astype(jnp.float32),
                         params["embed_mlp"], ss[0])
    cnts = _sc_counts()(recv3d, zerosc, onesc)

    for i, sp in enumerate(params["steps"]):
        s_buf, r_buf = _sc_gather()(node_lat, send3d, recv3d)
        edge_lat = _tc_edge(edge_lat, s_buf, r_buf,
                            sp["edge_mlp"], ss[1 + 2 * i])
        sums = _sc_scatter()(edge_lat, recv3d, zeros128)
        node_lat = _tc_node(node_lat, sums, cnts,
                            sp["node_mlp"], ss[2 + 2 * i])

    return node_lat[:, None, :]
